# Initial kernel scaffold; baseline (speedup 1.0000x reference)
import jax
import jax.numpy as jnp
from jax.experimental import pallas as pl


def kernel(x, weight, bias):
    raise NotImplementedError("write your pallas kernel here")


# speedup vs baseline: 1.0074x; 1.0074x over previous
import functools

import jax
import jax.numpy as jnp
from jax import lax
from jax.experimental import pallas as pl
from jax.experimental.pallas import tpu as pltpu


def _round_up(a, m):
    return (a + m - 1) // m * m


def _cnn_matmul_kernel(w2_ref, bias_ref, x_ref, out_ref, *, n_out, lout):
    xt = x_ref[...].astype(jnp.bfloat16)
    r = lax.dot_general(
        xt, w2_ref[...],
        dimension_numbers=(((1,), (0,)), ((), ())),
        preferred_element_type=jnp.float32,
    )

    n_pow2 = 1
    while n_pow2 * 2 <= lout:
        n_pow2 *= 2
    p = n_pow2 * n_out
    m = r[:, :p]
    while p > n_out:
        p //= 2
        m = jnp.maximum(m[:, :p], m[:, p:2 * p])
    for k in range(n_pow2, lout):
        m = jnp.maximum(m, r[:, k * n_out:(k + 1) * n_out])

    out_ref[...] = jnp.maximum(m + bias_ref[...], 0.0)


def _build_w2(weight, lout):
    n_out, n_in, ksize = weight.shape
    length = lout + ksize - 1
    t = jnp.arange(lout)
    l = jnp.arange(length)
    j = l[:, None] - t[None, :]
    valid = (j >= 0) & (j < ksize)
    jc = jnp.clip(j, 0, ksize - 1)
    wv = weight[:, :, jc]
    wv = jnp.where(valid[None, None], wv, 0.0)
    w2 = wv.transpose(1, 2, 3, 0).reshape(n_in * length, lout * n_out)
    return w2.astype(jnp.bfloat16)


@functools.partial(jax.jit, static_argnames=("block_b",))
def _forward(x, weight, bias, *, block_b=1024):
    B, C, L = x.shape
    O, Cw, K = weight.shape
    assert Cw == C and L >= K, (x.shape, weight.shape)
    lout = L - K + 1

    x2 = x.reshape(B, C * L)
    w2 = _build_w2(weight, lout)
    bias_row = bias.reshape(1, O).astype(jnp.float32)

    tb = min(block_b, _round_up(B, 8))
    b_pad = _round_up(B, tb)
    if b_pad != B:
        x2 = jnp.pad(x2, ((0, b_pad - B), (0, 0)))
    grid = b_pad // tb

    kernel_fn = functools.partial(_cnn_matmul_kernel, n_out=O, lout=lout)

    in_bytes = x2.size * x2.dtype.itemsize + w2.size * 2 + bias_row.size * 4
    out_bytes = b_pad * O * 4

    out = pl.pallas_call(
        kernel_fn,
        out_shape=jax.ShapeDtypeStruct((b_pad, O), jnp.float32),
        grid=(grid,),
        in_specs=[
            pl.BlockSpec((C * L, lout * O), lambda i: (0, 0)),
            pl.BlockSpec((1, O), lambda i: (0, 0)),
            pl.BlockSpec((tb, C * L), lambda i: (i, 0)),
        ],
        out_specs=pl.BlockSpec((tb, O), lambda i: (i, 0)),
        compiler_params=pltpu.CompilerParams(
            dimension_semantics=("parallel",)),
        cost_estimate=pl.CostEstimate(
            flops=2 * b_pad * lout * C * K * O,
            transcendentals=0,
            bytes_accessed=in_bytes + out_bytes),
    )(w2, bias_row, x2)

    return out[:B, :, None]


def kernel(x, weight, bias):
    return _forward(x, weight, bias)


# speedup vs baseline: 1.9990x; 1.9842x over previous
import functools

import jax
import jax.numpy as jnp
from jax import lax
from jax.experimental import pallas as pl
from jax.experimental.pallas import tpu as pltpu


def _round_up(a, m):
    return (a + m - 1) // m * m


def _cnn_mxu_kernel(w2_ref, bias_ref, x_ref, out_ref, *, n_out, lout):
    L, C, TB = x_ref.shape
    x2 = x_ref[...].reshape(L * C, TB).astype(jnp.bfloat16)
    r = lax.dot_general(
        w2_ref[...], x2,
        dimension_numbers=(((1,), (0,)), ((), ())),
        preferred_element_type=jnp.float32,
    )

    n_pow2 = 1
    while n_pow2 * 2 <= lout:
        n_pow2 *= 2
    p = n_pow2 * n_out
    m = r[:p]
    while p > n_out:
        p //= 2
        m = jnp.maximum(m[:p], m[p:2 * p])
    for k in range(n_pow2, lout):
        m = jnp.maximum(m, r[k * n_out:(k + 1) * n_out])

    out_ref[...] = jnp.maximum(m + bias_ref[...], 0.0)


def _build_w2(weight, lout):
    n_out, n_in, ksize = weight.shape
    length = lout + ksize - 1
    t = jnp.arange(lout)
    l = jnp.arange(length)
    j = l[:, None] - t[None, :]
    valid = (j >= 0) & (j < ksize)
    jc = jnp.clip(j, 0, ksize - 1)
    wv = weight[:, :, jc]
    wv = jnp.where(valid[None, None], wv, 0.0)
    w2 = wv.transpose(3, 0, 2, 1).reshape(lout * n_out, length * n_in)
    return w2.astype(jnp.bfloat16)


@functools.partial(jax.jit, static_argnames=("block_b",))
def _forward(x, weight, bias, *, block_b=512):
    B, C, L = x.shape
    O, Cw, K = weight.shape
    assert Cw == C and L >= K, (x.shape, weight.shape)
    lout = L - K + 1

    xt = jnp.transpose(x, (2, 1, 0))
    w2 = _build_w2(weight, lout)
    bias_col = bias.reshape(O, 1).astype(jnp.float32)

    tb = min(block_b, _round_up(B, 128))
    b_pad = _round_up(B, tb)
    if b_pad != B:
        xt = jnp.pad(xt, ((0, 0), (0, 0), (0, b_pad - B)))
    grid = b_pad // tb

    kernel_fn = functools.partial(_cnn_mxu_kernel, n_out=O, lout=lout)

    in_bytes = xt.size * xt.dtype.itemsize + w2.size * 2 + bias_col.size * 4
    out_bytes = O * b_pad * 4

    out = pl.pallas_call(
        kernel_fn,
        out_shape=jax.ShapeDtypeStruct((O, b_pad), jnp.float32),
        grid=(grid,),
        in_specs=[
            pl.BlockSpec((lout * O, L * C), lambda i: (0, 0)),
            pl.BlockSpec((O, 1), lambda i: (0, 0)),
            pl.BlockSpec((L, C, tb), lambda i: (0, 0, i)),
        ],
        out_specs=pl.BlockSpec((O, tb), lambda i: (0, i)),
        compiler_params=pltpu.CompilerParams(
            dimension_semantics=("parallel",)),
        cost_estimate=pl.CostEstimate(
            flops=2 * b_pad * lout * C * K * O,
            transcendentals=0,
            bytes_accessed=in_bytes + out_bytes),
    )(w2, bias_col, xt)

    return out[:, :B].T[:, :, None]


def kernel(x, weight, bias):
    return _forward(x, weight, bias)


# speedup vs baseline: 5.2104x; 2.6065x over previous
import functools

import jax
import jax.numpy as jnp
from jax import lax
from jax.experimental import pallas as pl
from jax.experimental.pallas import tpu as pltpu


def _round_up(a, m):
    return (a + m - 1) // m * m


def _cnn_mxu_kernel(w2_ref, bias_ref, x_ref, out_ref, *, n_out, lout):
    L, C, TB = x_ref.shape
    x2 = x_ref[...].reshape(L * C, TB).astype(jnp.bfloat16)
    r = lax.dot_general(
        w2_ref[...], x2,
        dimension_numbers=(((1,), (0,)), ((), ())),
        preferred_element_type=jnp.float32,
    )

    n_pow2 = 1
    while n_pow2 * 2 <= lout:
        n_pow2 *= 2
    p = n_pow2 * n_out
    m = r[:p]
    while p > n_out:
        p //= 2
        m = jnp.maximum(m[:p], m[p:2 * p])
    for k in range(n_pow2, lout):
        m = jnp.maximum(m, r[k * n_out:(k + 1) * n_out])

    out_ref[...] = jnp.maximum(m + bias_ref[...], 0.0)


def _build_w2(weight, lout):
    n_out, n_in, ksize = weight.shape
    length = lout + ksize - 1
    t = jnp.arange(lout)
    l = jnp.arange(length)
    j = l[:, None] - t[None, :]
    valid = (j >= 0) & (j < ksize)
    jc = jnp.clip(j, 0, ksize - 1)
    wv = weight[:, :, jc]
    wv = jnp.where(valid[None, None], wv, 0.0)
    w2 = wv.transpose(3, 0, 2, 1).reshape(lout * n_out, length * n_in)
    return w2.astype(jnp.bfloat16)


@functools.partial(jax.jit, static_argnames=("block_b",))
def _forward(x, weight, bias, *, block_b=2048):
    B, C, L = x.shape
    O, Cw, K = weight.shape
    assert Cw == C and L >= K, (x.shape, weight.shape)
    lout = L - K + 1

    xt = jnp.transpose(x, (2, 1, 0))
    w2 = _build_w2(weight, lout)
    bias_col = bias.reshape(O, 1).astype(jnp.float32)

    tb = min(block_b, _round_up(B, 128))
    b_pad = _round_up(B, tb)
    if b_pad != B:
        xt = jnp.pad(xt, ((0, 0), (0, 0), (0, b_pad - B)))
    grid = b_pad // tb

    kernel_fn = functools.partial(_cnn_mxu_kernel, n_out=O, lout=lout)

    in_bytes = xt.size * xt.dtype.itemsize + w2.size * 2 + bias_col.size * 4
    out_bytes = O * b_pad * 4

    out = pl.pallas_call(
        kernel_fn,
        out_shape=jax.ShapeDtypeStruct((O, b_pad), jnp.float32),
        grid=(grid,),
        in_specs=[
            pl.BlockSpec((lout * O, L * C), lambda i: (0, 0)),
            pl.BlockSpec((O, 1), lambda i: (0, 0)),
            pl.BlockSpec((L, C, tb), lambda i: (0, 0, i)),
        ],
        out_specs=pl.BlockSpec((O, tb), lambda i: (0, i)),
        compiler_params=pltpu.CompilerParams(
            dimension_semantics=("parallel",)),
        cost_estimate=pl.CostEstimate(
            flops=2 * b_pad * lout * C * K * O,
            transcendentals=0,
            bytes_accessed=in_bytes + out_bytes),
    )(w2, bias_col, xt)

    return out[:, :B].T[:, :, None]


def kernel(x, weight, bias):
    return _forward(x, weight, bias)


# speedup vs baseline: 8.7974x; 1.6884x over previous
import functools

import jax
import jax.numpy as jnp
from jax import lax
from jax.experimental import pallas as pl
from jax.experimental.pallas import tpu as pltpu


def _round_up(a, m):
    return (a + m - 1) // m * m


def _cnn_mxu_kernel(w2_ref, bias_ref, x_ref, out_ref, *, n_out, lout):
    L, C, TB = x_ref.shape
    x2 = x_ref[...].reshape(L * C, TB).astype(jnp.bfloat16)
    r = lax.dot_general(
        w2_ref[...], x2,
        dimension_numbers=(((1,), (0,)), ((), ())),
        preferred_element_type=jnp.float32,
    )

    n_pow2 = 1
    while n_pow2 * 2 <= lout:
        n_pow2 *= 2
    p = n_pow2 * n_out
    m = r[:p]
    while p > n_out:
        p //= 2
        m = jnp.maximum(m[:p], m[p:2 * p])
    for k in range(n_pow2, lout):
        m = jnp.maximum(m, r[k * n_out:(k + 1) * n_out])

    out_ref[...] = jnp.maximum(m + bias_ref[...], 0.0)


def _build_w2(weight, lout):
    n_out, n_in, ksize = weight.shape
    length = lout + ksize - 1
    t = jnp.arange(lout)
    l = jnp.arange(length)
    j = l[:, None] - t[None, :]
    valid = (j >= 0) & (j < ksize)
    jc = jnp.clip(j, 0, ksize - 1)
    wv = weight[:, :, jc]
    wv = jnp.where(valid[None, None], wv, 0.0)
    w2 = wv.transpose(3, 0, 2, 1).reshape(lout * n_out, length * n_in)
    return w2.astype(jnp.bfloat16)


@functools.partial(jax.jit, static_argnames=("block_b",))
def _forward(x, weight, bias, *, block_b=8192):
    B, C, L = x.shape
    O, Cw, K = weight.shape
    assert Cw == C and L >= K, (x.shape, weight.shape)
    lout = L - K + 1

    xt = jnp.transpose(x, (2, 1, 0))
    w2 = _build_w2(weight, lout)
    bias_col = bias.reshape(O, 1).astype(jnp.float32)

    tb = min(block_b, _round_up(B, 128))
    b_pad = _round_up(B, tb)
    if b_pad != B:
        xt = jnp.pad(xt, ((0, 0), (0, 0), (0, b_pad - B)))
    grid = b_pad // tb

    kernel_fn = functools.partial(_cnn_mxu_kernel, n_out=O, lout=lout)

    in_bytes = xt.size * xt.dtype.itemsize + w2.size * 2 + bias_col.size * 4
    out_bytes = O * b_pad * 4

    out = pl.pallas_call(
        kernel_fn,
        out_shape=jax.ShapeDtypeStruct((O, b_pad), jnp.float32),
        grid=(grid,),
        in_specs=[
            pl.BlockSpec((lout * O, L * C), lambda i: (0, 0)),
            pl.BlockSpec((O, 1), lambda i: (0, 0)),
            pl.BlockSpec((L, C, tb), lambda i: (0, 0, i)),
        ],
        out_specs=pl.BlockSpec((O, tb), lambda i: (0, i)),
        compiler_params=pltpu.CompilerParams(
            dimension_semantics=("parallel",)),
        cost_estimate=pl.CostEstimate(
            flops=2 * b_pad * lout * C * K * O,
            transcendentals=0,
            bytes_accessed=in_bytes + out_bytes),
    )(w2, bias_col, xt)

    return out[:, :B].T[:, :, None]


def kernel(x, weight, bias):
    return _forward(x, weight, bias)


# speedup vs baseline: 9.9232x; 1.1280x over previous
import functools

import jax
import jax.numpy as jnp
from jax import lax
from jax.experimental import pallas as pl
from jax.experimental.pallas import tpu as pltpu


def _round_up(a, m):
    return (a + m - 1) // m * m


def _cnn_mxu_kernel(w2_ref, bias_ref, x_ref, out_ref, *, n_out, lout):
    L, C, TB = x_ref.shape
    x2 = x_ref[...].reshape(L * C, TB).astype(jnp.bfloat16)
    r = lax.dot_general(
        w2_ref[...], x2,
        dimension_numbers=(((1,), (0,)), ((), ())),
        preferred_element_type=jnp.float32,
    )

    n_pow2 = 1
    while n_pow2 * 2 <= lout:
        n_pow2 *= 2
    p = n_pow2 * n_out
    m = r[:p]
    while p > n_out:
        p //= 2
        m = jnp.maximum(m[:p], m[p:2 * p])
    for k in range(n_pow2, lout):
        m = jnp.maximum(m, r[k * n_out:(k + 1) * n_out])

    out_ref[...] = jnp.maximum(m + bias_ref[...], 0.0)


def _build_w2(weight, lout):
    n_out, n_in, ksize = weight.shape
    length = lout + ksize - 1
    t = jnp.arange(lout)
    l = jnp.arange(length)
    j = l[:, None] - t[None, :]
    valid = (j >= 0) & (j < ksize)
    jc = jnp.clip(j, 0, ksize - 1)
    wv = weight[:, :, jc]
    wv = jnp.where(valid[None, None], wv, 0.0)
    w2 = wv.transpose(3, 0, 2, 1).reshape(lout * n_out, length * n_in)
    return w2.astype(jnp.bfloat16)


@functools.partial(jax.jit, static_argnames=("block_b",))
def _forward(x, weight, bias, *, block_b=16384):
    B, C, L = x.shape
    O, Cw, K = weight.shape
    assert Cw == C and L >= K, (x.shape, weight.shape)
    lout = L - K + 1

    xt = jnp.transpose(x, (2, 1, 0))
    w2 = _build_w2(weight, lout)
    bias_col = bias.reshape(O, 1).astype(jnp.float32)

    tb = min(block_b, _round_up(B, 128))
    b_pad = _round_up(B, tb)
    if b_pad != B:
        xt = jnp.pad(xt, ((0, 0), (0, 0), (0, b_pad - B)))
    grid = b_pad // tb

    kernel_fn = functools.partial(_cnn_mxu_kernel, n_out=O, lout=lout)

    in_bytes = xt.size * xt.dtype.itemsize + w2.size * 2 + bias_col.size * 4
    out_bytes = O * b_pad * 4

    out = pl.pallas_call(
        kernel_fn,
        out_shape=jax.ShapeDtypeStruct((O, b_pad), jnp.float32),
        grid=(grid,),
        in_specs=[
            pl.BlockSpec((lout * O, L * C), lambda i: (0, 0)),
            pl.BlockSpec((O, 1), lambda i: (0, 0)),
            pl.BlockSpec((L, C, tb), lambda i: (0, 0, i)),
        ],
        out_specs=pl.BlockSpec((O, tb), lambda i: (0, i)),
        compiler_params=pltpu.CompilerParams(
            dimension_semantics=("parallel",)),
        cost_estimate=pl.CostEstimate(
            flops=2 * b_pad * lout * C * K * O,
            transcendentals=0,
            bytes_accessed=in_bytes + out_bytes),
    )(w2, bias_col, xt)

    return out[:, :B].T[:, :, None]


def kernel(x, weight, bias):
    return _forward(x, weight, bias)


# speedup vs baseline: 10.1652x; 1.0244x over previous
import functools

import jax
import jax.numpy as jnp
from jax import lax
from jax.experimental import pallas as pl
from jax.experimental.pallas import tpu as pltpu


def _round_up(a, m):
    return (a + m - 1) // m * m


def _cnn_mxu_kernel(w2_ref, bias_ref, x_ref, out_ref, *, n_out, lout):
    L, C, TB = x_ref.shape
    x2 = x_ref[...].reshape(L * C, TB).astype(jnp.bfloat16)
    r = lax.dot_general(
        w2_ref[...], x2,
        dimension_numbers=(((1,), (0,)), ((), ())),
        preferred_element_type=jnp.float32,
    )

    n_pow2 = 1
    while n_pow2 * 2 <= lout:
        n_pow2 *= 2
    p = n_pow2 * n_out
    m = r[:p]
    while p > n_out:
        p //= 2
        m = jnp.maximum(m[:p], m[p:2 * p])
    for k in range(n_pow2, lout):
        m = jnp.maximum(m, r[k * n_out:(k + 1) * n_out])

    out_ref[...] = jnp.maximum(m + bias_ref[...], 0.0)


def _build_w2(weight, lout):
    n_out, n_in, ksize = weight.shape
    length = lout + ksize - 1
    t = jnp.arange(lout)
    l = jnp.arange(length)
    j = l[:, None] - t[None, :]
    valid = (j >= 0) & (j < ksize)
    jc = jnp.clip(j, 0, ksize - 1)
    wv = weight[:, :, jc]
    wv = jnp.where(valid[None, None], wv, 0.0)
    w2 = wv.transpose(3, 0, 2, 1).reshape(lout * n_out, length * n_in)
    return w2.astype(jnp.bfloat16)


@functools.partial(jax.jit, static_argnames=("block_b",))
def _forward(x, weight, bias, *, block_b=32768):
    B, C, L = x.shape
    O, Cw, K = weight.shape
    assert Cw == C and L >= K, (x.shape, weight.shape)
    lout = L - K + 1

    xt = jnp.transpose(x, (2, 1, 0))
    w2 = _build_w2(weight, lout)
    bias_col = bias.reshape(O, 1).astype(jnp.float32)

    tb = min(block_b, _round_up(B, 128))
    b_pad = _round_up(B, tb)
    if b_pad != B:
        xt = jnp.pad(xt, ((0, 0), (0, 0), (0, b_pad - B)))
    grid = b_pad // tb

    kernel_fn = functools.partial(_cnn_mxu_kernel, n_out=O, lout=lout)

    in_bytes = xt.size * xt.dtype.itemsize + w2.size * 2 + bias_col.size * 4
    out_bytes = O * b_pad * 4

    out = pl.pallas_call(
        kernel_fn,
        out_shape=jax.ShapeDtypeStruct((O, b_pad), jnp.float32),
        grid=(grid,),
        in_specs=[
            pl.BlockSpec((lout * O, L * C), lambda i: (0, 0)),
            pl.BlockSpec((O, 1), lambda i: (0, 0)),
            pl.BlockSpec((L, C, tb), lambda i: (0, 0, i)),
        ],
        out_specs=pl.BlockSpec((O, tb), lambda i: (0, i)),
        compiler_params=pltpu.CompilerParams(
            dimension_semantics=("parallel",)),
        cost_estimate=pl.CostEstimate(
            flops=2 * b_pad * lout * C * K * O,
            transcendentals=0,
            bytes_accessed=in_bytes + out_bytes),
    )(w2, bias_col, xt)

    return out[:, :B].T[:, :, None]


def kernel(x, weight, bias):
    return _forward(x, weight, bias)


# speedup vs baseline: 13.7060x; 1.3483x over previous
import functools

import jax
import jax.numpy as jnp
from jax import lax
from jax.experimental import pallas as pl
from jax.experimental.pallas import tpu as pltpu


def _round_up(a, m):
    return (a + m - 1) // m * m


def _cnn_mxu_kernel(w2_ref, bias_ref, x_ref, out_ref, *, n_out, lout):
    L, C, TB = x_ref.shape
    x2 = x_ref[...].reshape(L * C, TB).astype(jnp.bfloat16)
    r = lax.dot_general(
        w2_ref[...], x2,
        dimension_numbers=(((1,), (0,)), ((), ())),
        preferred_element_type=jnp.float32,
    )

    n_pow2 = 1
    while n_pow2 * 2 <= lout:
        n_pow2 *= 2
    p = n_pow2 * n_out
    m = r[:p]
    while p > n_out:
        p //= 2
        m = jnp.maximum(m[:p], m[p:2 * p])
    for k in range(n_pow2, lout):
        m = jnp.maximum(m, r[k * n_out:(k + 1) * n_out])

    out_ref[...] = jnp.maximum(m + bias_ref[...], 0.0)[:, None, :]


def _build_w2(weight, lout):
    n_out, n_in, ksize = weight.shape
    length = lout + ksize - 1
    t = jnp.arange(lout)
    l = jnp.arange(length)
    j = l[:, None] - t[None, :]
    valid = (j >= 0) & (j < ksize)
    jc = jnp.clip(j, 0, ksize - 1)
    wv = weight[:, :, jc]
    wv = jnp.where(valid[None, None], wv, 0.0)
    w2 = wv.transpose(3, 0, 2, 1).reshape(lout * n_out, length * n_in)
    return w2.astype(jnp.bfloat16)


@functools.partial(jax.jit, static_argnames=("block_b",))
def _forward(x, weight, bias, *, block_b=32768):
    B, C, L = x.shape
    O, Cw, K = weight.shape
    assert Cw == C and L >= K, (x.shape, weight.shape)
    lout = L - K + 1

    xt = jnp.transpose(x, (2, 1, 0))
    w2 = _build_w2(weight, lout)
    bias_col = bias.reshape(O, 1).astype(jnp.float32)

    tb = min(block_b, _round_up(B, 128))
    b_pad = _round_up(B, tb)
    if b_pad != B:
        xt = jnp.pad(xt, ((0, 0), (0, 0), (0, b_pad - B)))
    grid = b_pad // tb

    kernel_fn = functools.partial(_cnn_mxu_kernel, n_out=O, lout=lout)

    in_bytes = xt.size * xt.dtype.itemsize + w2.size * 2 + bias_col.size * 4
    out_bytes = O * b_pad * 4

    out = pl.pallas_call(
        kernel_fn,
        out_shape=jax.ShapeDtypeStruct((O, 1, b_pad), jnp.float32),
        grid=(grid,),
        in_specs=[
            pl.BlockSpec((lout * O, L * C), lambda i: (0, 0)),
            pl.BlockSpec((O, 1), lambda i: (0, 0)),
            pl.BlockSpec((L, C, tb), lambda i: (0, 0, i)),
        ],
        out_specs=pl.BlockSpec((O, 1, tb), lambda i: (0, 0, i)),
        compiler_params=pltpu.CompilerParams(
            dimension_semantics=("parallel",)),
        cost_estimate=pl.CostEstimate(
            flops=2 * b_pad * lout * C * K * O,
            transcendentals=0,
            bytes_accessed=in_bytes + out_bytes),
    )(w2, bias_col, xt)

    return jnp.transpose(out, (2, 0, 1))[:B]


def kernel(x, weight, bias):
    return _forward(x, weight, bias)
